# Initial kernel scaffold; baseline (speedup 1.0000x reference)
import jax
import jax.numpy as jnp
from jax.experimental import pallas as pl


def kernel(x, edge_index, batch, W1, b1, W2, b2, Wfc, bfc):
    raise NotImplementedError("write your pallas kernel here")


# speedup vs baseline: 33.5305x; 33.5305x over previous
import functools

import jax
import jax.numpy as jnp
from jax import lax
from jax.experimental import pallas as pl
from jax.experimental.pallas import tpu as pltpu
from jax.experimental.pallas import tpu_sc as plsc

N = 100000
E = 3200000
NG = 512

NC, NS = 2, 16
NW = NC * NS
CHUNK = 128
CPW = 800
EPAD = NW * CPW * CHUNK
RBD = 16
RBE = 8
NPAD = 100352
SL = NPAD // NS
BR = 2048
NBR = NPAD // BR

_mesh = plsc.VectorSubcoreMesh(
    core_axis_name="c", subcore_axis_name="s", num_cores=NC, num_subcores=NS)
_sc_params = pltpu.CompilerParams(use_tc_tiling_on_sc=False)


def _deg_body(dst_hbm, ones_hbm, zeros_hbm, out_hbm, idx_v, ones_v, acc, *, rb):
  cid = lax.axis_index("c")
  sid = lax.axis_index("s")
  wid = cid * NS + sid
  tlo = sid * SL
  pltpu.sync_copy(ones_hbm, ones_v)
  pltpu.sync_copy(zeros_hbm.at[pl.ds(tlo, SL)], acc.at[pl.ds(tlo, SL)])
  plsc.subcore_barrier()

  base = wid * CPW

  def blk(b, carry):
    r0 = base + b * rb
    pltpu.sync_copy(dst_hbm.at[pl.ds(r0, rb)], idx_v)
    for j in range(rb):
      pltpu.sync_copy(ones_v, acc.at[idx_v.at[j]], add=True)
    return carry

  lax.fori_loop(0, CPW // rb, blk, 0)
  plsc.subcore_barrier()
  pltpu.sync_copy(acc.at[pl.ds(tlo, SL)], out_hbm.at[cid, pl.ds(tlo, SL)])


_deg_call = pl.kernel(
    functools.partial(_deg_body, rb=RBD),
    out_type=jax.ShapeDtypeStruct((NC, NPAD, 8), jnp.float32),
    mesh=_mesh,
    compiler_params=_sc_params,
    scratch_types=[
        pltpu.VMEM((RBD, CHUNK), jnp.int32),
        pltpu.VMEM((CHUNK, 8), jnp.float32),
        pltpu.VMEM_SHARED((NPAD, 8), jnp.float32),
    ],
)


def _edge_body(src_hbm, dst_hbm, feat_hbm, zeros_hbm, out_hbm,
               srcv, dstv, rows, acc, gsem, *, f, rb):
  cid = lax.axis_index("c")
  sid = lax.axis_index("s")
  wid = cid * NS + sid
  tlo = sid * SL
  pltpu.sync_copy(zeros_hbm.at[pl.ds(tlo, SL)], acc.at[pl.ds(tlo, SL)])
  plsc.subcore_barrier()

  base = wid * CPW

  def blk(b, carry):
    r0 = base + b * rb
    pltpu.sync_copy(src_hbm.at[pl.ds(r0, rb)], srcv)
    pltpu.sync_copy(dst_hbm.at[pl.ds(r0, rb)], dstv)
    cps = [pltpu.async_copy(feat_hbm.at[srcv.at[j]], rows.at[j], gsem)
           for j in range(rb)]
    for cp in cps:
      cp.wait()
    for j in range(rb):
      pltpu.sync_copy(rows.at[j], acc.at[dstv.at[j]], add=True)
    return carry

  lax.fori_loop(0, CPW // rb, blk, 0)
  plsc.subcore_barrier()
  pltpu.sync_copy(acc.at[pl.ds(tlo, SL)], out_hbm.at[cid, pl.ds(tlo, SL)])


def _make_edge_call(f):
  return pl.kernel(
      functools.partial(_edge_body, f=f, rb=RBE),
      out_type=jax.ShapeDtypeStruct((NC, NPAD, f), jnp.float32),
      mesh=_mesh,
      compiler_params=_sc_params,
      scratch_types=[
          pltpu.VMEM((RBE, CHUNK), jnp.int32),
          pltpu.VMEM((RBE, CHUNK), jnp.int32),
          pltpu.VMEM((RBE, CHUNK, f), jnp.float32),
          pltpu.VMEM_SHARED((NPAD, f), jnp.float32),
          pltpu.SemaphoreType.DMA,
      ],
  )


_edge_call8 = _make_edge_call(8)
_edge_call16 = _make_edge_call(16)


def _b_body(d0, d1, xp, dis_o, xs_o):
  dis = lax.rsqrt(d0[...][:, :1] + d1[...][:, :1] + 1.0)
  dis_o[...] = dis
  xs_o[...] = xp[...] * dis


def _b_call(d0, d1, xp):
  return pl.pallas_call(
      _b_body,
      grid=(NBR,),
      in_specs=[
          pl.BlockSpec((BR, 8), lambda i: (i, 0)),
          pl.BlockSpec((BR, 8), lambda i: (i, 0)),
          pl.BlockSpec((BR, 8), lambda i: (i, 0)),
      ],
      out_specs=[
          pl.BlockSpec((BR, 1), lambda i: (i, 0)),
          pl.BlockSpec((BR, 8), lambda i: (i, 0)),
      ],
      out_shape=[
          jax.ShapeDtypeStruct((NPAD, 1), jnp.float32),
          jax.ShapeDtypeStruct((NPAD, 8), jnp.float32),
      ],
  )(d0, d1, xp)


def _d_body(s0, s1, xs, dis, w1, b1, w2, g2_o):
  t = (s0[...] + s1[...] + xs[...]) * dis[...]
  h1 = jnp.dot(t, w1[...], preferred_element_type=jnp.float32) + b1[...]
  h1 = jnp.maximum(h1, 0.0)
  g2_o[...] = jnp.dot(h1, w2[...], preferred_element_type=jnp.float32) * dis[...]


def _d_call(s0, s1, xs, dis, w1, b1, w2):
  return pl.pallas_call(
      _d_body,
      grid=(NBR,),
      in_specs=[
          pl.BlockSpec((BR, 8), lambda i: (i, 0)),
          pl.BlockSpec((BR, 8), lambda i: (i, 0)),
          pl.BlockSpec((BR, 8), lambda i: (i, 0)),
          pl.BlockSpec((BR, 1), lambda i: (i, 0)),
          pl.BlockSpec((8, 16), lambda i: (0, 0)),
          pl.BlockSpec((1, 16), lambda i: (0, 0)),
          pl.BlockSpec((16, 16), lambda i: (0, 0)),
      ],
      out_specs=pl.BlockSpec((BR, 16), lambda i: (i, 0)),
      out_shape=jax.ShapeDtypeStruct((NPAD, 16), jnp.float32),
  )(s0, s1, xs, dis, w1, b1, w2)


def _f_body(a0, a1, g2, dis, b2, bt, wfc, bfc, out, pooled, counts):
  i = pl.program_id(0)

  @pl.when(i == 0)
  def _init():
    pooled[...] = jnp.zeros_like(pooled)
    counts[...] = jnp.zeros_like(counts)

  h2 = (a0[...] + a1[...] + g2[...]) * dis[...] + b2[...]
  h2 = jnp.maximum(h2, 0.0)
  ids = lax.broadcasted_iota(jnp.int32, (NG, BR), 0)
  oh = (ids == bt[...]).astype(jnp.float32)
  pooled[...] += jnp.dot(oh, h2, preferred_element_type=jnp.float32)
  counts[...] += jnp.sum(oh, axis=1, keepdims=True)

  @pl.when(i == NBR - 1)
  def _fin():
    gmean = pooled[...] / jnp.maximum(counts[...], 1.0)
    out[...] = jnp.dot(gmean, wfc[...], preferred_element_type=jnp.float32) + bfc[...]


def _f_call(a0, a1, g2, dis, b2, bt, wfc, bfc):
  return pl.pallas_call(
      _f_body,
      grid=(NBR,),
      in_specs=[
          pl.BlockSpec((BR, 16), lambda i: (i, 0)),
          pl.BlockSpec((BR, 16), lambda i: (i, 0)),
          pl.BlockSpec((BR, 16), lambda i: (i, 0)),
          pl.BlockSpec((BR, 1), lambda i: (i, 0)),
          pl.BlockSpec((1, 16), lambda i: (0, 0)),
          pl.BlockSpec((1, BR), lambda i: (0, i)),
          pl.BlockSpec((16, 1), lambda i: (0, 0)),
          pl.BlockSpec((1, 1), lambda i: (0, 0)),
      ],
      out_specs=pl.BlockSpec((NG, 1), lambda i: (0, 0)),
      out_shape=jax.ShapeDtypeStruct((NG, 1), jnp.float32),
      scratch_shapes=[
          pltpu.VMEM((NG, 16), jnp.float32),
          pltpu.VMEM((NG, 1), jnp.float32),
      ],
  )(a0, a1, g2, dis, b2, bt, wfc, bfc)


def kernel(x, edge_index, batch, W1, b1, W2, b2, Wfc, bfc):
  src = edge_index[0].astype(jnp.int32)
  dst = edge_index[1].astype(jnp.int32)
  src2d = jnp.concatenate(
      [src, jnp.zeros((EPAD - E,), jnp.int32)]).reshape(-1, CHUNK)
  dst2d = jnp.concatenate(
      [dst, jnp.full((EPAD - E,), N, jnp.int32)]).reshape(-1, CHUNK)
  xpad = jnp.zeros((NPAD, 8), jnp.float32).at[:N, :2].set(x)
  w1pad = jnp.zeros((8, 16), jnp.float32).at[:2].set(W1)
  ones8 = jnp.ones((CHUNK, 8), jnp.float32)
  bt2d = jnp.concatenate(
      [batch.astype(jnp.int32),
       jnp.full((NPAD - N,), NG, jnp.int32)]).reshape(1, NPAD)
  z8 = jnp.zeros((NPAD, 8), jnp.float32)
  z16 = jnp.zeros((NPAD, 16), jnp.float32)

  deg2 = _deg_call(dst2d, ones8, z8)
  dis, xs = _b_call(deg2[0], deg2[1], xpad)
  s = _edge_call8(src2d, dst2d, xs, z8)
  g2 = _d_call(s[0], s[1], xs, dis, w1pad, b1.reshape(1, 16), W2)
  a = _edge_call16(src2d, dst2d, g2, z16)
  return _f_call(a[0], a[1], g2, dis, b2.reshape(1, 16), bt2d, Wfc,
                 bfc.reshape(1, 1))


# speedup vs baseline: 38.1762x; 1.1386x over previous
import functools

import jax
import jax.numpy as jnp
from jax import lax
from jax.experimental import pallas as pl
from jax.experimental.pallas import tpu as pltpu
from jax.experimental.pallas import tpu_sc as plsc

N = 100000
E = 3200000
NG = 512

NC, NS = 2, 16
NW = NC * NS
CHUNK = 128
CPW = 800
EPAD = NW * CPW * CHUNK
RBD = 16
RBE = 4
NPAD = 100352
SL = NPAD // NS
BR = 2048
NBR = NPAD // BR

_mesh = plsc.VectorSubcoreMesh(
    core_axis_name="c", subcore_axis_name="s", num_cores=NC, num_subcores=NS)
_sc_params = pltpu.CompilerParams(use_tc_tiling_on_sc=False)


def _deg_body(dst_hbm, ones_hbm, zeros_hbm, out_hbm, idx_v, ones_v, acc, *, rb):
  cid = lax.axis_index("c")
  sid = lax.axis_index("s")
  wid = cid * NS + sid
  tlo = sid * SL
  pltpu.sync_copy(ones_hbm, ones_v)
  pltpu.sync_copy(zeros_hbm.at[pl.ds(tlo, SL)], acc.at[pl.ds(tlo, SL)])
  plsc.subcore_barrier()

  base = wid * CPW

  def blk(b, carry):
    r0 = base + b * rb
    pltpu.sync_copy(dst_hbm.at[pl.ds(r0, rb)], idx_v)
    for j in range(rb):
      pltpu.sync_copy(ones_v, acc.at[idx_v.at[j]], add=True)
    return carry

  lax.fori_loop(0, CPW // rb, blk, 0)
  plsc.subcore_barrier()
  pltpu.sync_copy(acc.at[pl.ds(tlo, SL)], out_hbm.at[cid, pl.ds(tlo, SL)])


_deg_call = pl.kernel(
    functools.partial(_deg_body, rb=RBD),
    out_type=jax.ShapeDtypeStruct((NC, NPAD, 8), jnp.float32),
    mesh=_mesh,
    compiler_params=_sc_params,
    scratch_types=[
        pltpu.VMEM((RBD, CHUNK), jnp.int32),
        pltpu.VMEM((CHUNK, 8), jnp.float32),
        pltpu.VMEM_SHARED((NPAD, 8), jnp.float32),
    ],
)


def _edge_body(ei_hbm, feat_hbm, zeros_hbm, out_hbm,
               idxv, rows, acc, gsem0, gsem1, ssem, *, f, rb):
  cid = lax.axis_index("c")
  sid = lax.axis_index("s")
  wid = cid * NS + sid
  tlo = sid * SL
  pltpu.sync_copy(zeros_hbm.at[pl.ds(tlo, SL)], acc.at[pl.ds(tlo, SL)])
  plsc.subcore_barrier()

  base = wid * CPW
  gsems = (gsem0, gsem1)

  def blk(it, carry):
    r0 = base + it * (2 * rb)
    gs = []
    for b in range(2):
      pltpu.sync_copy(ei_hbm.at[pl.ds(r0 + b * rb, rb)], idxv.at[b])
      gs.append([pltpu.async_copy(feat_hbm.at[idxv.at[b, j, 0]],
                                  rows.at[b, j], gsems[b])
                 for j in range(rb)])
    ss = []
    for b in range(2):
      for cp in gs[b]:
        cp.wait()
      ss += [pltpu.async_copy(rows.at[b, j], acc.at[idxv.at[b, j, 1]],
                              ssem, add=True) for j in range(rb)]
    for cp in ss:
      cp.wait()
    return carry

  lax.fori_loop(0, CPW // (2 * rb), blk, 0)
  plsc.subcore_barrier()
  pltpu.sync_copy(acc.at[pl.ds(tlo, SL)], out_hbm.at[cid, pl.ds(tlo, SL)])


def _make_edge_call(f):
  return pl.kernel(
      functools.partial(_edge_body, f=f, rb=RBE),
      out_type=jax.ShapeDtypeStruct((NC, NPAD, f), jnp.float32),
      mesh=_mesh,
      compiler_params=_sc_params,
      scratch_types=[
          pltpu.VMEM((2, RBE, 2, CHUNK), jnp.int32),
          pltpu.VMEM((2, RBE, CHUNK, f), jnp.float32),
          pltpu.VMEM_SHARED((NPAD, f), jnp.float32),
          pltpu.SemaphoreType.DMA,
          pltpu.SemaphoreType.DMA,
          pltpu.SemaphoreType.DMA,
      ],
  )


_edge_call8 = _make_edge_call(8)
_edge_call16 = _make_edge_call(16)


def _b_body(d0, d1, xp, dis_o, xs_o):
  dis = lax.rsqrt(d0[...][:, :1] + d1[...][:, :1] + 1.0)
  dis_o[...] = dis
  xs_o[...] = xp[...] * dis


def _b_call(d0, d1, xp):
  return pl.pallas_call(
      _b_body,
      grid=(NBR,),
      in_specs=[
          pl.BlockSpec((BR, 8), lambda i: (i, 0)),
          pl.BlockSpec((BR, 8), lambda i: (i, 0)),
          pl.BlockSpec((BR, 8), lambda i: (i, 0)),
      ],
      out_specs=[
          pl.BlockSpec((BR, 1), lambda i: (i, 0)),
          pl.BlockSpec((BR, 8), lambda i: (i, 0)),
      ],
      out_shape=[
          jax.ShapeDtypeStruct((NPAD, 1), jnp.float32),
          jax.ShapeDtypeStruct((NPAD, 8), jnp.float32),
      ],
  )(d0, d1, xp)


def _d_body(s0, s1, xs, dis, w1, b1, w2, g2_o):
  t = (s0[...] + s1[...] + xs[...]) * dis[...]
  h1 = jnp.dot(t, w1[...], preferred_element_type=jnp.float32) + b1[...]
  h1 = jnp.maximum(h1, 0.0)
  g2_o[...] = jnp.dot(h1, w2[...], preferred_element_type=jnp.float32) * dis[...]


def _d_call(s0, s1, xs, dis, w1, b1, w2):
  return pl.pallas_call(
      _d_body,
      grid=(NBR,),
      in_specs=[
          pl.BlockSpec((BR, 8), lambda i: (i, 0)),
          pl.BlockSpec((BR, 8), lambda i: (i, 0)),
          pl.BlockSpec((BR, 8), lambda i: (i, 0)),
          pl.BlockSpec((BR, 1), lambda i: (i, 0)),
          pl.BlockSpec((8, 16), lambda i: (0, 0)),
          pl.BlockSpec((1, 16), lambda i: (0, 0)),
          pl.BlockSpec((16, 16), lambda i: (0, 0)),
      ],
      out_specs=pl.BlockSpec((BR, 16), lambda i: (i, 0)),
      out_shape=jax.ShapeDtypeStruct((NPAD, 16), jnp.float32),
  )(s0, s1, xs, dis, w1, b1, w2)


def _f_body(a0, a1, g2, dis, b2, bt, wfc, bfc, out, pooled, counts):
  i = pl.program_id(0)

  @pl.when(i == 0)
  def _init():
    pooled[...] = jnp.zeros_like(pooled)
    counts[...] = jnp.zeros_like(counts)

  h2 = (a0[...] + a1[...] + g2[...]) * dis[...] + b2[...]
  h2 = jnp.maximum(h2, 0.0)
  ids = lax.broadcasted_iota(jnp.int32, (NG, BR), 0)
  oh = (ids == bt[...]).astype(jnp.float32)
  pooled[...] += jnp.dot(oh, h2, preferred_element_type=jnp.float32)
  counts[...] += jnp.sum(oh, axis=1, keepdims=True)

  @pl.when(i == NBR - 1)
  def _fin():
    gmean = pooled[...] / jnp.maximum(counts[...], 1.0)
    out[...] = jnp.dot(gmean, wfc[...], preferred_element_type=jnp.float32) + bfc[...]


def _f_call(a0, a1, g2, dis, b2, bt, wfc, bfc):
  return pl.pallas_call(
      _f_body,
      grid=(NBR,),
      in_specs=[
          pl.BlockSpec((BR, 16), lambda i: (i, 0)),
          pl.BlockSpec((BR, 16), lambda i: (i, 0)),
          pl.BlockSpec((BR, 16), lambda i: (i, 0)),
          pl.BlockSpec((BR, 1), lambda i: (i, 0)),
          pl.BlockSpec((1, 16), lambda i: (0, 0)),
          pl.BlockSpec((1, BR), lambda i: (0, i)),
          pl.BlockSpec((16, 1), lambda i: (0, 0)),
          pl.BlockSpec((1, 1), lambda i: (0, 0)),
      ],
      out_specs=pl.BlockSpec((NG, 1), lambda i: (0, 0)),
      out_shape=jax.ShapeDtypeStruct((NG, 1), jnp.float32),
      scratch_shapes=[
          pltpu.VMEM((NG, 16), jnp.float32),
          pltpu.VMEM((NG, 1), jnp.float32),
      ],
  )(a0, a1, g2, dis, b2, bt, wfc, bfc)


def kernel(x, edge_index, batch, W1, b1, W2, b2, Wfc, bfc):
  src = edge_index[0].astype(jnp.int32)
  dst = edge_index[1].astype(jnp.int32)
  src2d = jnp.concatenate(
      [src, jnp.zeros((EPAD - E,), jnp.int32)]).reshape(-1, CHUNK)
  dst2d = jnp.concatenate(
      [dst, jnp.full((EPAD - E,), N, jnp.int32)]).reshape(-1, CHUNK)
  xpad = jnp.zeros((NPAD, 8), jnp.float32).at[:N, :2].set(x)
  w1pad = jnp.zeros((8, 16), jnp.float32).at[:2].set(W1)
  ones8 = jnp.ones((CHUNK, 8), jnp.float32)
  bt2d = jnp.concatenate(
      [batch.astype(jnp.int32),
       jnp.full((NPAD - N,), NG, jnp.int32)]).reshape(1, NPAD)
  z8 = jnp.zeros((NPAD, 8), jnp.float32)
  z16 = jnp.zeros((NPAD, 16), jnp.float32)

  ei3d = jnp.stack([src2d, dst2d], axis=1)
  deg2 = _deg_call(dst2d, ones8, z8)
  dis, xs = _b_call(deg2[0], deg2[1], xpad)
  s = _edge_call8(ei3d, xs, z8)
  g2 = _d_call(s[0], s[1], xs, dis, w1pad, b1.reshape(1, 16), W2)
  a = _edge_call16(ei3d, g2, z16)
  return _f_call(a[0], a[1], g2, dis, b2.reshape(1, 16), bt2d, Wfc,
                 bfc.reshape(1, 1))
